# baseline (speedup 1.0000x reference)
import functools

import jax
import jax.numpy as jnp
from jax import lax
from jax.experimental import pallas as pl
from jax.experimental.pallas import tpu as pltpu
from jax.experimental.pallas import tpu_sc as plsc

NUM_ROWS = 1000000
DIM = 32
TOTAL = NUM_ROWS * DIM
NUM_WORKERS = 32
PER_W = TOTAL // NUM_WORKERS
CHUNK = 25000
N_CHUNKS = PER_W // CHUNK


def _sc_copy_body(table_hbm, out_hbm, buf0, buf1, is0, is1, os0, os1):
    wid = lax.axis_index("s") * 2 + lax.axis_index("c")
    base = wid * PER_W
    bufs = (buf0, buf1)
    isems = (is0, is1)
    osems = (os0, os1)

    pltpu.async_copy(table_hbm.at[pl.ds(base, CHUNK)], buf0, is0)
    for i in range(N_CHUNKS):
        b = i % 2
        pltpu.make_async_copy(
            table_hbm.at[pl.ds(base + i * CHUNK, CHUNK)], bufs[b], isems[b]
        ).wait()
        if i >= 1:
            pltpu.make_async_copy(
                bufs[1 - b],
                out_hbm.at[pl.ds(base + (i - 1) * CHUNK, CHUNK)],
                osems[1 - b],
            ).wait()
        if i + 1 < N_CHUNKS:
            pltpu.async_copy(
                table_hbm.at[pl.ds(base + (i + 1) * CHUNK, CHUNK)],
                bufs[1 - b],
                isems[1 - b],
            )
        pltpu.async_copy(
            bufs[b], out_hbm.at[pl.ds(base + i * CHUNK, CHUNK)], osems[b]
        )
    lastb = (N_CHUNKS - 1) % 2
    pltpu.make_async_copy(
        bufs[lastb],
        out_hbm.at[pl.ds(base + (N_CHUNKS - 1) * CHUNK, CHUNK)],
        osems[lastb],
    ).wait()


def kernel(table):
    sc_copy = pl.kernel(
        _sc_copy_body,
        mesh=plsc.VectorSubcoreMesh(core_axis_name="c", subcore_axis_name="s"),
        out_type=jax.ShapeDtypeStruct((TOTAL,), jnp.float32),
        scratch_types=[
            pltpu.VMEM((CHUNK,), jnp.float32),
            pltpu.VMEM((CHUNK,), jnp.float32),
            pltpu.SemaphoreType.DMA,
            pltpu.SemaphoreType.DMA,
            pltpu.SemaphoreType.DMA,
            pltpu.SemaphoreType.DMA,
        ],
    )
    flat = table.reshape(TOTAL)
    return sc_copy(flat).reshape(NUM_ROWS, DIM)


# speedup vs baseline: 1.1518x; 1.1518x over previous
import functools

import jax
import jax.numpy as jnp
from jax import lax
from jax.experimental import pallas as pl
from jax.experimental.pallas import tpu as pltpu
from jax.experimental.pallas import tpu_sc as plsc

NUM_ROWS = 1000000
DIM = 32
CHUNK = 504
N_CHUNKS = 62
BASE_UNITS = 31248 // 8


def _sc_copy_body(table_hbm, out_hbm, buf0, buf1, tail, is0, is1, os0, os1, ts):
    wid = lax.axis_index("s") * 2 + lax.axis_index("c")
    base = pl.multiple_of((BASE_UNITS * wid + wid // 4) * 8, 8)
    bufs = (buf0, buf1)
    isems = (is0, is1)
    osems = (os0, os1)

    pltpu.async_copy(table_hbm.at[pl.ds(base, CHUNK)], buf0, is0)
    for i in range(N_CHUNKS):
        b = i % 2
        pltpu.make_async_copy(
            table_hbm.at[pl.ds(base + i * CHUNK, CHUNK)], bufs[b], isems[b]
        ).wait()
        if i >= 1:
            pltpu.make_async_copy(
                bufs[1 - b],
                out_hbm.at[pl.ds(base + (i - 1) * CHUNK, CHUNK)],
                osems[1 - b],
            ).wait()
        if i + 1 < N_CHUNKS:
            pltpu.async_copy(
                table_hbm.at[pl.ds(base + (i + 1) * CHUNK, CHUNK)],
                bufs[1 - b],
                isems[1 - b],
            )
        pltpu.async_copy(
            bufs[b], out_hbm.at[pl.ds(base + i * CHUNK, CHUNK)], osems[b]
        )

    @pl.when(wid % 4 == 3)
    def _():
        tb = pl.multiple_of(base + N_CHUNKS * CHUNK, 8)
        pltpu.make_async_copy(table_hbm.at[pl.ds(tb, 8)], tail, ts).start()
        pltpu.make_async_copy(table_hbm.at[pl.ds(tb, 8)], tail, ts).wait()
        pltpu.make_async_copy(tail, out_hbm.at[pl.ds(tb, 8)], ts).start()
        pltpu.make_async_copy(tail, out_hbm.at[pl.ds(tb, 8)], ts).wait()

    lastb = (N_CHUNKS - 1) % 2
    pltpu.make_async_copy(
        bufs[lastb],
        out_hbm.at[pl.ds(base + (N_CHUNKS - 1) * CHUNK, CHUNK)],
        osems[lastb],
    ).wait()


def kernel(table):
    sc_copy = pl.kernel(
        _sc_copy_body,
        mesh=plsc.VectorSubcoreMesh(core_axis_name="c", subcore_axis_name="s"),
        out_type=jax.ShapeDtypeStruct((NUM_ROWS, DIM), jnp.float32),
        scratch_types=[
            pltpu.VMEM((CHUNK, DIM), jnp.float32),
            pltpu.VMEM((CHUNK, DIM), jnp.float32),
            pltpu.VMEM((8, DIM), jnp.float32),
            pltpu.SemaphoreType.DMA,
            pltpu.SemaphoreType.DMA,
            pltpu.SemaphoreType.DMA,
            pltpu.SemaphoreType.DMA,
            pltpu.SemaphoreType.DMA,
        ],
    )
    return sc_copy(table)


# speedup vs baseline: 1.1531x; 1.0011x over previous
import functools

import jax
import jax.numpy as jnp
from jax import lax
from jax.experimental import pallas as pl
from jax.experimental.pallas import tpu as pltpu
from jax.experimental.pallas import tpu_sc as plsc

NUM_ROWS = 1000000
DIM = 32
CHUNK = 504
N_CHUNKS = 62
BASE_UNITS = 31248 // 8


def _sc_copy_body(table_hbm, out_hbm, buf0, buf1, tail, is0, is1, os0, os1, ts):
    wid = lax.axis_index("s") * 2 + lax.axis_index("c")
    base = pl.multiple_of((BASE_UNITS * wid + wid // 4) * 8, 8)
    bufs = (buf0, buf1)
    isems = (is0, is1)
    osems = (os0, os1)

    pltpu.async_copy(table_hbm.at[pl.ds(base, CHUNK)], buf0, is0)
    for i in range(N_CHUNKS):
        b = i % 2
        pltpu.make_async_copy(
            table_hbm.at[pl.ds(base + i * CHUNK, CHUNK)], bufs[b], isems[b]
        ).wait()
        if i >= 1:
            pltpu.make_async_copy(
                bufs[1 - b],
                out_hbm.at[pl.ds(base + (i - 1) * CHUNK, CHUNK)],
                osems[1 - b],
            ).wait()
        if i + 1 < N_CHUNKS:
            pltpu.async_copy(
                table_hbm.at[pl.ds(base + (i + 1) * CHUNK, CHUNK)],
                bufs[1 - b],
                isems[1 - b],
            )
        pltpu.async_copy(
            bufs[b], out_hbm.at[pl.ds(base + i * CHUNK, CHUNK)], osems[b]
        )

    @pl.when(wid % 4 == 3)
    def _():
        tb = pl.multiple_of(base + N_CHUNKS * CHUNK, 8)
        pltpu.make_async_copy(table_hbm.at[pl.ds(tb, 8)], tail, ts).start()
        pltpu.make_async_copy(table_hbm.at[pl.ds(tb, 8)], tail, ts).wait()
        pltpu.make_async_copy(tail, out_hbm.at[pl.ds(tb, 8)], ts).start()
        pltpu.make_async_copy(tail, out_hbm.at[pl.ds(tb, 8)], ts).wait()

    lastb = (N_CHUNKS - 1) % 2
    pltpu.make_async_copy(
        bufs[lastb],
        out_hbm.at[pl.ds(base + (N_CHUNKS - 1) * CHUNK, CHUNK)],
        osems[lastb],
    ).wait()


def kernel(table):
    sc_copy = pl.kernel(
        _sc_copy_body,
        mesh=plsc.VectorSubcoreMesh(core_axis_name="c", subcore_axis_name="s"),
        compiler_params=pltpu.CompilerParams(use_tc_tiling_on_sc=True),
        out_type=jax.ShapeDtypeStruct((NUM_ROWS, DIM), jnp.float32),
        scratch_types=[
            pltpu.VMEM((CHUNK, DIM), jnp.float32),
            pltpu.VMEM((CHUNK, DIM), jnp.float32),
            pltpu.VMEM((8, DIM), jnp.float32),
            pltpu.SemaphoreType.DMA,
            pltpu.SemaphoreType.DMA,
            pltpu.SemaphoreType.DMA,
            pltpu.SemaphoreType.DMA,
            pltpu.SemaphoreType.DMA,
        ],
    )
    return sc_copy(table)


# speedup vs baseline: 1.2251x; 1.0624x over previous
import jax
import jax.numpy as jnp
from jax.experimental import pallas as pl
from jax.experimental.pallas import tpu as pltpu

NUM_ROWS = 1000000
DIM = 32
BLOCK = 8000
GRID = NUM_ROWS // BLOCK


def _copy_kernel(x_ref, o_ref):
    o_ref[...] = x_ref[...]


def kernel(table):
    return pl.pallas_call(
        _copy_kernel,
        grid=(GRID,),
        in_specs=[pl.BlockSpec((BLOCK, DIM), lambda i: (i, 0))],
        out_specs=pl.BlockSpec((BLOCK, DIM), lambda i: (i, 0)),
        out_shape=jax.ShapeDtypeStruct((NUM_ROWS, DIM), table.dtype),
    )(table)


# speedup vs baseline: 12.3026x; 10.0418x over previous
import jax
import jax.numpy as jnp
from jax.experimental import pallas as pl
from jax.experimental.pallas import tpu as pltpu

NUM_ROWS = 1000000
DIM = 32
BLK = 16384
GRID = (NUM_ROWS + BLK - 1) // BLK


def _copy_kernel(x_ref, o_ref):
    o_ref[...] = x_ref[...]


def kernel(table):
    t = table.T
    out = pl.pallas_call(
        _copy_kernel,
        grid=(GRID,),
        in_specs=[pl.BlockSpec((DIM, BLK), lambda i: (0, i))],
        out_specs=pl.BlockSpec((DIM, BLK), lambda i: (0, i)),
        out_shape=jax.ShapeDtypeStruct((DIM, NUM_ROWS), table.dtype),
    )(t)
    return out.T
